# baseline (speedup 1.0000x reference)
import dataclasses

import jax
import jax.numpy as jnp
from jax import lax
from jax.experimental import pallas as pl
from jax.experimental.pallas import tpu as pltpu
from jax.experimental.pallas import tpu_sc as plsc

_NC = 2
_NS = 16
_NW = _NC * _NS
_K = 128
_L = 16


def kernel(indexes, table):
    num_indices = indexes.shape[0]
    vocab, dim = table.shape
    pack = 128 // dim
    b_per_w = num_indices // _NW
    nchunk = b_per_w // _K

    idx = indexes.astype(jnp.int32)
    idx4 = (idx // pack).reshape(_NW * nchunk, _K)
    colb = ((idx % pack) * dim).reshape(_NW, b_per_w)
    table2 = table.reshape(vocab // pack, dim * pack)

    mesh = plsc.VectorSubcoreMesh(core_axis_name="c", subcore_axis_name="s")
    cp = pltpu.CompilerParams()
    if "needs_layout_passes" in pltpu.CompilerParams.__dataclass_fields__:
        cp = dataclasses.replace(cp, needs_layout_passes=False)

    @jax.jit
    def run(table2_arr, idx4_arr, colb_arr):
        @pl.kernel(
            out_type=jax.ShapeDtypeStruct((num_indices, dim), table.dtype),
            mesh=mesh,
            compiler_params=cp,
            scratch_types=[
                pltpu.VMEM((nchunk, _K), jnp.int32),
                pltpu.VMEM((b_per_w,), jnp.int32),
                pltpu.VMEM((2, _K, dim * pack), jnp.float32),
                pltpu.VMEM((2, _K, dim), jnp.float32),
                pltpu.SemaphoreType.DMA,
                pltpu.SemaphoreType.DMA,
                pltpu.SemaphoreType.DMA,
                pltpu.SemaphoreType.DMA,
                pltpu.SemaphoreType.DMA,
            ],
        )
        def gather_kernel(
            table_hbm, idx4_hbm, colb_hbm, out_hbm,
            idx4_v, colb_v, rows_v, out_v, isem, gsem0, gsem1, wsem0, wsem1,
        ):
            wid = lax.axis_index("s") * _NC + lax.axis_index("c")
            pltpu.async_copy(
                idx4_hbm.at[pl.ds(wid * nchunk, nchunk)], idx4_v, isem
            )
            pltpu.async_copy(colb_hbm.at[wid], colb_v, isem)
            pltpu.make_async_copy(
                idx4_hbm.at[pl.ds(0, nchunk)], idx4_v, isem
            ).wait()
            pltpu.make_async_copy(colb_hbm.at[0], colb_v, isem).wait()

            gsems = (gsem0, gsem1)
            wsems = (wsem0, wsem1)
            iota = lax.iota(jnp.int32, _L)

            def fire_gather(j):
                return pltpu.async_copy(
                    table_hbm.at[idx4_v.at[j]],
                    rows_v.at[j % 2],
                    gsems[j % 2],
                )

            def select(j):
                rows_j = rows_v.at[j % 2]
                out_j = out_v.at[j % 2]

                @pl.loop(0, _K // _L)
                def _(b):
                    base = b * _L
                    rowv = iota + base
                    rem16 = colb_v[pl.ds(j * _K + base, _L)]
                    for c in range(dim):
                        vals = plsc.load_gather(rows_j, [rowv, rem16 + c])
                        plsc.store_scatter(
                            out_j,
                            [rowv, jnp.full((_L,), c, jnp.int32)],
                            vals,
                        )

            gathers = [fire_gather(0), fire_gather(1)]
            writes = []
            for j in range(nchunk):
                if j >= 2:
                    writes[j - 2].wait()
                gathers[j].wait()
                select(j)
                if j + 2 < nchunk:
                    gathers.append(fire_gather(j + 2))
                writes.append(
                    pltpu.async_copy(
                        out_v.at[j % 2],
                        out_hbm.at[pl.ds(wid * b_per_w + j * _K, _K)],
                        wsems[j % 2],
                    )
                )
            writes[-2].wait()
            writes[-1].wait()

        return gather_kernel(table2_arr, idx4_arr, colb_arr)

    return run(table2, idx4, colb)


# speedup vs baseline: 1.6972x; 1.6972x over previous
import jax
import jax.numpy as jnp
from jax import lax
from jax.experimental import pallas as pl
from jax.experimental.pallas import tpu as pltpu
from jax.experimental.pallas import tpu_sc as plsc

_NC = 2
_NS = 16
_NW = _NC * _NS
_NSEM = 8


def kernel(indexes, table):
    num_indices = indexes.shape[0]
    dim = table.shape[1]
    b_per_w = num_indices // _NW
    idx = indexes.astype(jnp.int32).reshape(_NW, b_per_w)

    mesh = plsc.VectorSubcoreMesh(core_axis_name="c", subcore_axis_name="s")

    @jax.jit
    def run(table_arr, idx_arr):
        @pl.kernel(
            out_type=jax.ShapeDtypeStruct((num_indices, dim), table_arr.dtype),
            mesh=mesh,
            scratch_types=[
                pltpu.VMEM((b_per_w,), jnp.int32),
                pltpu.VMEM((b_per_w, dim), jnp.float32),
                pltpu.SemaphoreType.DMA,
            ]
            + [pltpu.SemaphoreType.DMA] * _NSEM,
        )
        def gather_kernel(
            table_hbm, idx_hbm, out_hbm, idx_v, rows_v, isem, *sems
        ):
            wid = lax.axis_index("s") * _NC + lax.axis_index("c")
            pltpu.async_copy(idx_hbm.at[wid], idx_v, isem).wait()

            @pl.loop(0, b_per_w // 16)
            def _(j):
                base = j * 16
                v16 = idx_v[pl.ds(base, 16)]
                for k in range(16):
                    pltpu.async_copy(
                        table_hbm.at[pl.ds(v16[k], 1)],
                        rows_v.at[pl.ds(base + k, 1)],
                        sems[k % _NSEM],
                    )

            rows_per_sem = b_per_w // _NSEM
            for s in range(_NSEM):
                pltpu.make_async_copy(
                    table_hbm.at[pl.ds(0, rows_per_sem)],
                    rows_v.at[pl.ds(0, rows_per_sem)],
                    sems[s],
                ).wait()
            pltpu.sync_copy(rows_v, out_hbm.at[pl.ds(wid * b_per_w, b_per_w)])

        return gather_kernel(table_arr, idx_arr)

    return run(table, idx)
